# speedup vs baseline: 2.7973x; 2.7973x over previous
import jax
import jax.numpy as jnp
from jax import lax
from jax.experimental import pallas as pl

_NUM_CLASSES = 80
_A = 3
_C5 = 5 + _NUM_CLASSES
_H = 52
_W = 52
_HW = _H * _W
_STRIDE = 8.0
_ANCHOR_W = (10.0, 16.0, 33.0)
_ANCHOR_H = (13.0, 30.0, 23.0)


def _decode_body(x_ref, o_ref):
    a = pl.program_id(1)
    p = x_ref[0, 0]

    s = jax.nn.sigmoid(p)

    ii = lax.broadcasted_iota(jnp.int32, (1, _HW), 1)
    gxf = (ii % _W).astype(jnp.float32)
    gyf = (ii // _W).astype(jnp.float32)

    bx = (s[0:1, :] + gxf) * _STRIDE
    by = (s[1:2, :] + gyf) * _STRIDE

    aw = jnp.where(a == 0, _ANCHOR_W[0],
                   jnp.where(a == 1, _ANCHOR_W[1], _ANCHOR_W[2]))
    ah = jnp.where(a == 0, _ANCHOR_H[0],
                   jnp.where(a == 1, _ANCHOR_H[1], _ANCHOR_H[2]))
    bw = jnp.exp(p[2:3, :]) * aw
    bh = jnp.exp(p[3:4, :]) * ah

    obj = s[4:5, :]
    scores = s[5:, :] * obj

    out = jnp.concatenate([bx, by, bw, bh, obj, scores], axis=0)
    o_ref[0, 0] = out.T


def kernel(x):
    B = x.shape[0]
    xr = x.reshape(B, _A, _C5, _HW)
    out = pl.pallas_call(
        _decode_body,
        grid=(B, _A),
        in_specs=[pl.BlockSpec((1, 1, _C5, _HW), lambda b, a: (b, a, 0, 0))],
        out_specs=pl.BlockSpec((1, 1, _HW, _C5), lambda b, a: (b, a, 0, 0)),
        out_shape=jax.ShapeDtypeStruct((B, _A, _HW, _C5), jnp.float32),
    )(xr)
    return out.reshape(B, _A * _HW, _C5)


# speedup vs baseline: 4.0988x; 1.4653x over previous
import jax
import jax.numpy as jnp
from jax import lax
from jax.experimental import pallas as pl

_NUM_CLASSES = 80
_A = 3
_C5 = 5 + _NUM_CLASSES
_H = 52
_W = 52
_HW = _H * _W
_STRIDE = 8.0
_ANCHOR_W = (10.0, 16.0, 33.0)
_ANCHOR_H = (13.0, 30.0, 23.0)


def _decode_body(x_ref, o_ref):
    a = pl.program_id(1)
    p = x_ref[0].reshape(_C5, _HW)

    s = jax.nn.sigmoid(p)

    ii = lax.broadcasted_iota(jnp.int32, (1, _HW), 1)
    gxf = (ii % _W).astype(jnp.float32)
    gyf = (ii // _W).astype(jnp.float32)

    bx = (s[0:1, :] + gxf) * _STRIDE
    by = (s[1:2, :] + gyf) * _STRIDE

    aw = jnp.where(a == 0, _ANCHOR_W[0],
                   jnp.where(a == 1, _ANCHOR_W[1], _ANCHOR_W[2]))
    ah = jnp.where(a == 0, _ANCHOR_H[0],
                   jnp.where(a == 1, _ANCHOR_H[1], _ANCHOR_H[2]))
    bw = jnp.exp(p[2:3, :]) * aw
    bh = jnp.exp(p[3:4, :]) * ah

    obj = s[4:5, :]
    scores = s[5:, :] * obj

    out = jnp.concatenate([bx, by, bw, bh, obj, scores], axis=0)
    o_ref[0] = out.T


def kernel(x):
    B = x.shape[0]
    return pl.pallas_call(
        _decode_body,
        grid=(B, _A),
        in_specs=[pl.BlockSpec((1, _C5, _H, _W), lambda b, a: (b, a, 0, 0))],
        out_specs=pl.BlockSpec((1, _HW, _C5), lambda b, a: (b, a, 0)),
        out_shape=jax.ShapeDtypeStruct((B, _A * _HW, _C5), jnp.float32),
    )(x)
